# baseline (speedup 1.0000x reference)
import jax
import jax.numpy as jnp
from jax import lax
from jax.experimental import pallas as pl
from jax.experimental.pallas import tpu as pltpu
from jax.experimental.pallas import tpu_sc as plsc

B = 1024
S = 200
EMB = 128
NC = 2
NS = 16
NW = NC * NS
ROWS = B * S
RPW = ROWS // NW
SEQS = RPW // S
G = 4
P = 40
CR = G * P
PBLK = S // P
GBLK = SEQS // G
CHUNKS = PBLK * GBLK
NBUF = 2
ROUNDS = CHUNKS // NBUF
VPR = EMB // 16


def _body(x_hbm, table_hbm, pe_hbm, out_hbm,
          idx_v, pe_v, bufs, obufs, gsems, osems):
  wid = lax.axis_index("s") * NC + lax.axis_index("c")
  base = wid * RPW

  pltpu.sync_copy(x_hbm.at[pl.ds(base, RPW)], idx_v)
  pltpu.sync_copy(pe_hbm.at[pl.ds(0, S)], pe_v)

  def chunk_coords(j):
    g = lax.div(j, PBLK)
    pb = lax.rem(j, PBLK)
    return g, pb

  def start_gathers(j, slot):
    g, pb = chunk_coords(j)
    for q in range(G):
      off = (g * G + q) * S + pb * P
      pltpu.async_copy(
          table_hbm.at[idx_v.at[pl.ds(off, P)]],
          bufs[slot].at[pl.ds(q * P, P)], gsems[slot])

  def wait_gathers(slot):
    pltpu.make_async_copy(
        table_hbm.at[pl.ds(0, CR)], bufs[slot], gsems[slot]).wait()

  def start_outs(j, slot):
    g, pb = chunk_coords(j)
    for q in range(G):
      off = (g * G + q) * S + pb * P
      pltpu.async_copy(
          obufs[slot].at[pl.ds(q * P, P)],
          out_hbm.at[pl.ds(base + off, P)], osems[slot])

  def wait_outs(slot):
    pltpu.make_async_copy(
        obufs[slot], out_hbm.at[pl.ds(base, CR)], osems[slot]).wait()

  def add_pe(j, slot):
    buf = bufs[slot]
    obuf = obufs[slot]
    _, pb = chunk_coords(j)
    poff = pb * P

    def row(p, _):
      pp = poff + p
      pes = [pe_v[pp, pl.ds(c * 16, 16)] for c in range(VPR)]
      for q in range(G):
        r = q * P + p
        a = [buf[r, pl.ds(c * 16, 16)] for c in range(VPR)]
        for c in range(VPR):
          obuf[r, pl.ds(c * 16, 16)] = a[c] + pes[c]
      return 0

    lax.fori_loop(0, P, row, 0, unroll=2)

  for s in range(NBUF):
    start_gathers(s, s)

  def round_body(r, _):
    for s in range(NBUF):
      j = r * NBUF + s

      @pl.when(r >= 1)
      def _():
        wait_outs(s)
      wait_gathers(s)
      add_pe(j, s)

      @pl.when(r < ROUNDS - 1)
      def _():
        start_gathers(j + NBUF, s)
      start_outs(j, s)
    return 0

  lax.fori_loop(0, ROUNDS, round_body, 0)

  for s in range(NBUF):
    wait_outs(s)


def _kernel_body(x_hbm, table_hbm, pe_hbm, out_hbm, idx_v, pe_v, *rest):
  bufs = list(rest[:NBUF])
  obufs = list(rest[NBUF:2 * NBUF])
  gsems = list(rest[2 * NBUF:3 * NBUF])
  osems = list(rest[3 * NBUF:4 * NBUF])
  _body(x_hbm, table_hbm, pe_hbm, out_hbm, idx_v, pe_v,
        bufs, obufs, gsems, osems)


@jax.jit
def _run(x_flat, table, pe):
  buf_t = pltpu.VMEM((CR, EMB), jnp.float32)
  kern = pl.kernel(
      _kernel_body,
      out_type=jax.ShapeDtypeStruct((ROWS, EMB), jnp.float32),
      mesh=plsc.VectorSubcoreMesh(core_axis_name="c", subcore_axis_name="s"),
      scratch_types=(
          [pltpu.VMEM((RPW,), jnp.int32),
           pltpu.VMEM((S, EMB), jnp.float32)]
          + [buf_t] * (2 * NBUF)
          + [pltpu.SemaphoreType.DMA] * (2 * NBUF)
      ),
      name="pos_word_embedding_sc",
  )
  return kern(x_flat, table, pe)


def kernel(x, table, pe):
  b, s = x.shape
  out = _run(x.reshape(-1), table, pe)
  return out.reshape(b, s, EMB)
